# baseline (speedup 1.0000x reference)
import functools

import jax
import jax.numpy as jnp
from jax import lax
from jax.experimental import pallas as pl
from jax.experimental.pallas import tpu as pltpu
from jax.experimental.pallas import tpu_sc as plsc

N = 16384
C = 1000
GAMMA = 2.0
IGNORE_ID = -1

N_SC = 8192

NC = 2
NS = 16
NW = NC * NS
RPW = N_SC // NW
RBLK = 128
NRB = RPW // RBLK
NGR = RBLK // 16
CCH = 200
NCH = C // CCH
STEPS = NRB * NCH

BRT = 512
NB_TC = (N - N_SC) // BRT

_LN2 = 0.6931471805599453
_NEG = -3.0e38


def _ln(s):
    bits = lax.bitcast_convert_type(s, jnp.int32)
    e = ((bits >> 23) & 0xFF) - 127
    mbits = (bits & 0x007FFFFF) | 0x3F800000
    m = lax.bitcast_convert_type(mbits, jnp.float32)
    big = m > 1.4142135381698608
    m = jnp.where(big, m * 0.5, m)
    e = e + jnp.where(big, 1, 0)
    z = (m - 1.0) / (m + 1.0)
    z2 = z * z
    poly = 1.0 + z2 * (0.3333333432674408 + z2 * (0.20000000298023224 + z2 * 0.14285714924335480))
    lnm = 2.0 * z * poly
    return e.astype(jnp.float32) * _LN2 + lnm


def _sc_body(xT_hbm, t_hbm, a_hbm, out_hbm, bufs, tgt, alo, ob, sem):
    wid = lax.axis_index("s") * NC + lax.axis_index("c")
    row0 = wid * RPW
    iota = lax.iota(jnp.int32, 16)

    pltpu.sync_copy(t_hbm.at[pl.ds(row0, RPW)], tgt)
    pltpu.sync_copy(a_hbm, alo)

    def step_slices(s):
        rblk = lax.div(s, NCH)
        ch = s - rblk * NCH
        cbase = ch * CCH
        rbase = row0 + rblk * RBLK
        return rblk, ch, cbase, rbase

    def start_step(s, par):
        _, _, cbase, rbase = step_slices(s)
        pltpu.async_copy(
            xT_hbm.at[pl.ds(cbase, CCH), pl.ds(rbase, RBLK)],
            bufs.at[par],
            sem.at[par],
        )

    def wait_step(s, par):
        _, _, cbase, rbase = step_slices(s)
        pltpu.make_async_copy(
            xT_hbm.at[pl.ds(cbase, CCH), pl.ds(rbase, RBLK)],
            bufs.at[par],
            sem.at[par],
        ).wait()

    def step_body(s, carry):
        ms, ss, xs, acc, cnt = carry
        par = lax.rem(s, 2)
        rblk, ch, cbase, _ = step_slices(s)
        wait_step(s, par)

        @pl.when(s + 1 < STEPS)
        def _():
            start_step(s + 1, 1 - par)

        buf = bufs.at[par]

        def p1(c, mvs):
            return tuple(
                jnp.maximum(mvs[g], buf[c, pl.ds(g * 16, 16)])
                for g in range(NGR)
            )

        cms = lax.fori_loop(
            0, CCH, p1,
            tuple(jnp.full((16,), _NEG, jnp.float32) for _ in range(NGR)),
            unroll=4,
        )

        def p2(c, svs):
            return tuple(
                svs[g] + jnp.exp(buf[c, pl.ds(g * 16, 16)] - cms[g])
                for g in range(NGR)
            )

        css = lax.fori_loop(
            0, CCH, p2,
            tuple(jnp.zeros((16,), jnp.float32) for _ in range(NGR)),
            unroll=4,
        )

        ms_n, ss_n, xs_n = [], [], []
        for g in range(NGR):
            nm = jnp.maximum(ms[g], cms[g])
            ns = ss[g] * jnp.exp(ms[g] - nm) + css[g] * jnp.exp(cms[g] - nm)
            t = tgt[pl.ds(pl.multiple_of(rblk * RBLK + g * 16, 16), 16)]
            ts = jnp.where(t != IGNORE_ID, t, 0)
            inchunk = (ts >= cbase) & (ts < cbase + CCH)
            idx = jnp.clip(ts - cbase, 0, CCH - 1)
            gx = plsc.load_gather(buf, [idx, g * 16 + iota])
            nx = jnp.where(inchunk, gx, xs[g])
            ms_n.append(nm)
            ss_n.append(ns)
            xs_n.append(nx)

        def epilogue(args):
            ms_e, ss_e, xs_e, acc_e, cnt_e = args
            for g in range(NGR):
                t = tgt[pl.ds(pl.multiple_of(rblk * RBLK + g * 16, 16), 16)]
                valid = t != IGNORE_ID
                ts = jnp.where(valid, t, 0)
                logp = (xs_e[g] - ms_e[g]) - _ln(ss_e[g])
                p = jnp.exp(logp)
                av = plsc.load_gather(alo, [ts])
                om = 1.0 - p
                loss = -av * om * om * logp
                acc_e = acc_e + jnp.where(valid, loss, 0.0)
                cnt_e = cnt_e + jnp.where(valid, 1.0, 0.0)
            z = jnp.zeros((16,), jnp.float32)
            neg = jnp.full((16,), _NEG, jnp.float32)
            return (
                tuple(neg for _ in range(NGR)),
                tuple(z for _ in range(NGR)),
                tuple(z for _ in range(NGR)),
                acc_e,
                cnt_e,
            )

        return lax.cond(
            ch == NCH - 1,
            epilogue,
            lambda args: args,
            (tuple(ms_n), tuple(ss_n), tuple(xs_n), acc, cnt),
        )

    z16 = jnp.zeros((16,), jnp.float32)
    neg16 = jnp.full((16,), _NEG, jnp.float32)
    init = (
        tuple(neg16 for _ in range(NGR)),
        tuple(z16 for _ in range(NGR)),
        tuple(z16 for _ in range(NGR)),
        z16,
        z16,
    )
    start_step(0, 0)
    _, _, _, acc, cnt = lax.fori_loop(0, STEPS, step_body, init)
    ob[0, :] = acc
    ob[1, :] = cnt
    pltpu.sync_copy(ob, out_hbm.at[wid])


def _focal_sc(xT, targets, alpha_flat):
    mesh = plsc.VectorSubcoreMesh(core_axis_name="c", subcore_axis_name="s")
    f = pl.kernel(
        _sc_body,
        out_type=jax.ShapeDtypeStruct((NW, 2, 16), jnp.float32),
        mesh=mesh,
        compiler_params=pltpu.CompilerParams(needs_layout_passes=False),
        scratch_types=[
            pltpu.VMEM((2, CCH, RBLK), jnp.float32),
            pltpu.VMEM((RPW,), jnp.int32),
            pltpu.VMEM((C,), jnp.float32),
            pltpu.VMEM((2, 16), jnp.float32),
            pltpu.SemaphoreType.DMA((2,)),
        ],
    )
    return f(xT, targets, alpha_flat)


def _tc_body(x_ref, t_ref, a_ref, loss_ref, cnt_ref):
    xb = x_ref[...]
    t = t_ref[0, 0, :]
    valid = t != IGNORE_ID
    ts = jnp.where(valid, t, 0)
    cls = lax.broadcasted_iota(jnp.int32, (C, BRT), 0)
    onehot = cls == ts[None, :]
    rmax = jnp.max(xb, axis=0)
    ex = jnp.exp(xb - rmax[None, :])
    s = jnp.sum(ex, axis=0)
    xt = jnp.sum(jnp.where(onehot, xb, 0.0), axis=0)
    ab = jnp.broadcast_to(a_ref[:, 0:1], (C, BRT))
    at = jnp.sum(jnp.where(onehot, ab, 0.0), axis=0)
    logp = (xt - rmax) - jnp.log(s)
    p = jnp.exp(logp)
    om = 1.0 - p
    loss = -at * om * om * logp
    vf = valid.astype(jnp.float32)
    loss_ref[0, 0, 0] = jnp.sum(loss * vf)
    cnt_ref[0, 0, 0] = jnp.sum(vf)


def _focal_tc(xT, t3d, alpha_col):
    return pl.pallas_call(
        _tc_body,
        grid=(NB_TC,),
        in_specs=[
            pl.BlockSpec((C, BRT), lambda i: (0, i + N_SC // BRT)),
            pl.BlockSpec((1, 1, BRT), lambda i: (i, 0, 0)),
            pl.BlockSpec((C, 128), lambda i: (0, 0)),
        ],
        out_specs=[
            pl.BlockSpec((1, 1, 1), lambda i: (i, 0, 0), memory_space=pltpu.SMEM),
            pl.BlockSpec((1, 1, 1), lambda i: (i, 0, 0), memory_space=pltpu.SMEM),
        ],
        out_shape=[
            jax.ShapeDtypeStruct((NB_TC, 1, 1), jnp.float32),
            jax.ShapeDtypeStruct((NB_TC, 1, 1), jnp.float32),
        ],
    )(xT, t3d, alpha_col)


@jax.jit
def kernel(inputs, targets, alpha):
    alpha_flat = alpha.reshape(C)
    xT = inputs.T
    t3d = targets[N_SC:].reshape(NB_TC, 1, BRT)
    alpha_col = jnp.broadcast_to(alpha_flat[:, None], (C, 128))
    out = _focal_sc(xT, targets, alpha_flat)
    loss_tc, cnt_tc = _focal_tc(xT, t3d, alpha_col)
    loss_sum = out[:, 0, :].sum() + loss_tc.sum()
    cnt = out[:, 1, :].sum() + cnt_tc.sum()
    return loss_sum / jnp.maximum(cnt, 1.0)


# speedup vs baseline: 1.0245x; 1.0245x over previous
import functools

import jax
import jax.numpy as jnp
from jax import lax
from jax.experimental import pallas as pl
from jax.experimental.pallas import tpu as pltpu
from jax.experimental.pallas import tpu_sc as plsc

N = 16384
C = 1000
GAMMA = 2.0
IGNORE_ID = -1

N_SC = 8192

NC = 2
NS = 16
NW = NC * NS
RPW = N_SC // NW
RBLK = 128
NRB = RPW // RBLK
NGR = RBLK // 16
CCH = 200
NCH = C // CCH
STEPS = NRB * NCH

BRT = 512
NB_TC = (N - N_SC) // BRT

_LN2 = 0.6931471805599453
_NEG = -3.0e38


def _ln(s):
    bits = lax.bitcast_convert_type(s, jnp.int32)
    e = ((bits >> 23) & 0xFF) - 127
    mbits = (bits & 0x007FFFFF) | 0x3F800000
    m = lax.bitcast_convert_type(mbits, jnp.float32)
    big = m > 1.4142135381698608
    m = jnp.where(big, m * 0.5, m)
    e = e + jnp.where(big, 1, 0)
    z = (m - 1.0) / (m + 1.0)
    z2 = z * z
    poly = 1.0 + z2 * (0.3333333432674408 + z2 * (0.20000000298023224 + z2 * 0.14285714924335480))
    lnm = 2.0 * z * poly
    return e.astype(jnp.float32) * _LN2 + lnm


def _sc_body(xT_hbm, t_hbm, a_hbm, out_hbm, bufs, tgt, alo, ob, sem):
    wid = lax.axis_index("s") * NC + lax.axis_index("c")
    row0 = wid * RPW
    iota = lax.iota(jnp.int32, 16)

    pltpu.sync_copy(t_hbm.at[pl.ds(row0, RPW)], tgt)
    pltpu.sync_copy(a_hbm, alo)

    def step_slices(s):
        rblk = lax.div(s, NCH)
        ch = s - rblk * NCH
        cbase = ch * CCH
        rbase = row0 + rblk * RBLK
        return rblk, ch, cbase, rbase

    def start_step(s, par):
        _, _, cbase, rbase = step_slices(s)
        pltpu.async_copy(
            xT_hbm.at[pl.ds(cbase, CCH), pl.ds(rbase, RBLK)],
            bufs.at[par],
            sem.at[par],
        )

    def wait_step(s, par):
        _, _, cbase, rbase = step_slices(s)
        pltpu.make_async_copy(
            xT_hbm.at[pl.ds(cbase, CCH), pl.ds(rbase, RBLK)],
            bufs.at[par],
            sem.at[par],
        ).wait()

    def step_body(s, carry):
        ms, ss, xs, acc, cnt = carry
        par = lax.rem(s, 2)
        rblk, ch, cbase, _ = step_slices(s)
        wait_step(s, par)

        @pl.when(s + 1 < STEPS)
        def _():
            start_step(s + 1, 1 - par)

        buf = bufs.at[par]

        def p1(c, mvs):
            return tuple(
                jnp.maximum(mvs[g], buf[c, pl.ds(g * 16, 16)])
                for g in range(NGR)
            )

        cms = lax.fori_loop(
            0, CCH, p1,
            tuple(jnp.full((16,), _NEG, jnp.float32) for _ in range(NGR)),
            unroll=2,
        )

        def p2(c, svs):
            return tuple(
                svs[g] + jnp.exp(buf[c, pl.ds(g * 16, 16)] - cms[g])
                for g in range(NGR)
            )

        css = lax.fori_loop(
            0, CCH, p2,
            tuple(jnp.zeros((16,), jnp.float32) for _ in range(NGR)),
            unroll=2,
        )

        ms_n, ss_n, xs_n = [], [], []
        for g in range(NGR):
            nm = jnp.maximum(ms[g], cms[g])
            ns = ss[g] * jnp.exp(ms[g] - nm) + css[g] * jnp.exp(cms[g] - nm)
            t = tgt[pl.ds(pl.multiple_of(rblk * RBLK + g * 16, 16), 16)]
            ts = jnp.where(t != IGNORE_ID, t, 0)
            inchunk = (ts >= cbase) & (ts < cbase + CCH)
            idx = jnp.clip(ts - cbase, 0, CCH - 1)
            gx = plsc.load_gather(buf, [idx, g * 16 + iota])
            nx = jnp.where(inchunk, gx, xs[g])
            ms_n.append(nm)
            ss_n.append(ns)
            xs_n.append(nx)

        def epilogue(args):
            ms_e, ss_e, xs_e, acc_e, cnt_e = args
            for g in range(NGR):
                t = tgt[pl.ds(pl.multiple_of(rblk * RBLK + g * 16, 16), 16)]
                valid = t != IGNORE_ID
                ts = jnp.where(valid, t, 0)
                logp = (xs_e[g] - ms_e[g]) - _ln(ss_e[g])
                p = jnp.exp(logp)
                av = plsc.load_gather(alo, [ts])
                om = 1.0 - p
                loss = -av * om * om * logp
                acc_e = acc_e + jnp.where(valid, loss, 0.0)
                cnt_e = cnt_e + jnp.where(valid, 1.0, 0.0)
            z = jnp.zeros((16,), jnp.float32)
            neg = jnp.full((16,), _NEG, jnp.float32)
            return (
                tuple(neg for _ in range(NGR)),
                tuple(z for _ in range(NGR)),
                tuple(z for _ in range(NGR)),
                acc_e,
                cnt_e,
            )

        return lax.cond(
            ch == NCH - 1,
            epilogue,
            lambda args: args,
            (tuple(ms_n), tuple(ss_n), tuple(xs_n), acc, cnt),
        )

    z16 = jnp.zeros((16,), jnp.float32)
    neg16 = jnp.full((16,), _NEG, jnp.float32)
    init = (
        tuple(neg16 for _ in range(NGR)),
        tuple(z16 for _ in range(NGR)),
        tuple(z16 for _ in range(NGR)),
        z16,
        z16,
    )
    start_step(0, 0)
    _, _, _, acc, cnt = lax.fori_loop(0, STEPS, step_body, init)
    ob[0, :] = acc
    ob[1, :] = cnt
    pltpu.sync_copy(ob, out_hbm.at[wid])


def _focal_sc(xT, targets, alpha_flat):
    mesh = plsc.VectorSubcoreMesh(core_axis_name="c", subcore_axis_name="s")
    f = pl.kernel(
        _sc_body,
        out_type=jax.ShapeDtypeStruct((NW, 2, 16), jnp.float32),
        mesh=mesh,
        compiler_params=pltpu.CompilerParams(needs_layout_passes=False),
        scratch_types=[
            pltpu.VMEM((2, CCH, RBLK), jnp.float32),
            pltpu.VMEM((RPW,), jnp.int32),
            pltpu.VMEM((C,), jnp.float32),
            pltpu.VMEM((2, 16), jnp.float32),
            pltpu.SemaphoreType.DMA((2,)),
        ],
    )
    return f(xT, targets, alpha_flat)


def _tc_body(x_ref, t_ref, a_ref, loss_ref, cnt_ref):
    xb = x_ref[...]
    t = t_ref[0, 0, :]
    valid = t != IGNORE_ID
    ts = jnp.where(valid, t, 0)
    cls = lax.broadcasted_iota(jnp.int32, (C, BRT), 0)
    onehot = cls == ts[None, :]
    rmax = jnp.max(xb, axis=0)
    ex = jnp.exp(xb - rmax[None, :])
    s = jnp.sum(ex, axis=0)
    xt = jnp.sum(jnp.where(onehot, xb, 0.0), axis=0)
    ab = jnp.broadcast_to(a_ref[:, 0:1], (C, BRT))
    at = jnp.sum(jnp.where(onehot, ab, 0.0), axis=0)
    logp = (xt - rmax) - jnp.log(s)
    p = jnp.exp(logp)
    om = 1.0 - p
    loss = -at * om * om * logp
    vf = valid.astype(jnp.float32)
    loss_ref[0, 0, 0] = jnp.sum(loss * vf)
    cnt_ref[0, 0, 0] = jnp.sum(vf)


def _focal_tc(xT, t3d, alpha_col):
    return pl.pallas_call(
        _tc_body,
        grid=(NB_TC,),
        in_specs=[
            pl.BlockSpec((C, BRT), lambda i: (0, i + N_SC // BRT)),
            pl.BlockSpec((1, 1, BRT), lambda i: (i, 0, 0)),
            pl.BlockSpec((C, 128), lambda i: (0, 0)),
        ],
        out_specs=[
            pl.BlockSpec((1, 1, 1), lambda i: (i, 0, 0), memory_space=pltpu.SMEM),
            pl.BlockSpec((1, 1, 1), lambda i: (i, 0, 0), memory_space=pltpu.SMEM),
        ],
        out_shape=[
            jax.ShapeDtypeStruct((NB_TC, 1, 1), jnp.float32),
            jax.ShapeDtypeStruct((NB_TC, 1, 1), jnp.float32),
        ],
    )(xT, t3d, alpha_col)


@jax.jit
def kernel(inputs, targets, alpha):
    alpha_flat = alpha.reshape(C)
    xT = inputs.T
    t3d = targets[N_SC:].reshape(NB_TC, 1, BRT)
    alpha_col = jnp.broadcast_to(alpha_flat[:, None], (C, 128))
    out = _focal_sc(xT, targets, alpha_flat)
    loss_tc, cnt_tc = _focal_tc(xT, t3d, alpha_col)
    loss_sum = out[:, 0, :].sum() + loss_tc.sum()
    cnt = out[:, 1, :].sum() + cnt_tc.sum()
    return loss_sum / jnp.maximum(cnt, 1.0)


# speedup vs baseline: 1.0282x; 1.0036x over previous
import functools

import jax
import jax.numpy as jnp
from jax import lax
from jax.experimental import pallas as pl
from jax.experimental.pallas import tpu as pltpu
from jax.experimental.pallas import tpu_sc as plsc

N = 16384
C = 1000
GAMMA = 2.0
IGNORE_ID = -1

N_SC = 8192

NC = 2
NS = 16
NW = NC * NS
RPW = N_SC // NW
RBLK = 128
NRB = RPW // RBLK
NGR = RBLK // 16
CCH = 200
NCH = C // CCH
STEPS = NRB * NCH

BRT = 512
NB_TC = (N - N_SC) // BRT

_LN2 = 0.6931471805599453
_NEG = -3.0e38


def _ln(s):
    bits = lax.bitcast_convert_type(s, jnp.int32)
    e = ((bits >> 23) & 0xFF) - 127
    mbits = (bits & 0x007FFFFF) | 0x3F800000
    m = lax.bitcast_convert_type(mbits, jnp.float32)
    big = m > 1.4142135381698608
    m = jnp.where(big, m * 0.5, m)
    e = e + jnp.where(big, 1, 0)
    z = (m - 1.0) / (m + 1.0)
    z2 = z * z
    poly = 1.0 + z2 * (0.3333333432674408 + z2 * (0.20000000298023224 + z2 * 0.14285714924335480))
    lnm = 2.0 * z * poly
    return e.astype(jnp.float32) * _LN2 + lnm


def _sc_body(xT_hbm, t_hbm, a_hbm, out_hbm, bufs, tgt, alo, ob, sem):
    wid = lax.axis_index("s") * NC + lax.axis_index("c")
    row0 = wid * RPW
    iota = lax.iota(jnp.int32, 16)

    pltpu.sync_copy(t_hbm.at[pl.ds(row0, RPW)], tgt)
    pltpu.sync_copy(a_hbm, alo)

    def step_slices(s):
        rblk = lax.div(s, NCH)
        ch = s - rblk * NCH
        cbase = ch * CCH
        rbase = row0 + rblk * RBLK
        return rblk, ch, cbase, rbase

    def start_step(s, par):
        _, _, cbase, rbase = step_slices(s)
        pltpu.async_copy(
            xT_hbm.at[pl.ds(cbase, CCH), pl.ds(rbase, RBLK)],
            bufs.at[par],
            sem.at[par],
        )

    def wait_step(s, par):
        _, _, cbase, rbase = step_slices(s)
        pltpu.make_async_copy(
            xT_hbm.at[pl.ds(cbase, CCH), pl.ds(rbase, RBLK)],
            bufs.at[par],
            sem.at[par],
        ).wait()

    def step_body(s, carry):
        ms, ss, xs, acc, cnt = carry
        par = lax.rem(s, 2)
        rblk, ch, cbase, _ = step_slices(s)
        wait_step(s, par)

        @pl.when(s + 1 < STEPS)
        def _():
            start_step(s + 1, 1 - par)

        buf = bufs.at[par]

        SLAB = 25
        NSLAB = CCH // SLAB

        def slab_body(j, carry):
            cms_c, css_c = carry
            base = j * SLAB
            cms_o, css_o = [], []
            for g in range(NGR):
                vals = [
                    buf[base + k, pl.ds(g * 16, 16)] for k in range(SLAB)
                ]
                parts = list(vals)
                while len(parts) > 1:
                    parts = [
                        jnp.maximum(parts[q], parts[q + 1])
                        for q in range(0, len(parts) - 1, 2)
                    ] + ([parts[-1]] if len(parts) % 2 else [])
                sm = parts[0]
                sacc = [None, None, None]
                for k, v in enumerate(vals):
                    e = jnp.exp(v - sm)
                    a = sacc[k % 3]
                    sacc[k % 3] = e if a is None else a + e
                ssum = (sacc[0] + sacc[1]) + sacc[2]
                nm = jnp.maximum(cms_c[g], sm)
                ns = css_c[g] * jnp.exp(cms_c[g] - nm) + ssum * jnp.exp(sm - nm)
                cms_o.append(nm)
                css_o.append(ns)
            return tuple(cms_o), tuple(css_o)

        cms, css = lax.fori_loop(
            0, NSLAB, slab_body,
            (
                tuple(jnp.full((16,), _NEG, jnp.float32) for _ in range(NGR)),
                tuple(jnp.zeros((16,), jnp.float32) for _ in range(NGR)),
            ),
        )

        ms_n, ss_n, xs_n = [], [], []
        for g in range(NGR):
            nm = jnp.maximum(ms[g], cms[g])
            ns = ss[g] * jnp.exp(ms[g] - nm) + css[g] * jnp.exp(cms[g] - nm)
            t = tgt[pl.ds(pl.multiple_of(rblk * RBLK + g * 16, 16), 16)]
            ts = jnp.where(t != IGNORE_ID, t, 0)
            inchunk = (ts >= cbase) & (ts < cbase + CCH)
            idx = jnp.clip(ts - cbase, 0, CCH - 1)
            gx = plsc.load_gather(buf, [idx, g * 16 + iota])
            nx = jnp.where(inchunk, gx, xs[g])
            ms_n.append(nm)
            ss_n.append(ns)
            xs_n.append(nx)

        def epilogue(args):
            ms_e, ss_e, xs_e, acc_e, cnt_e = args
            for g in range(NGR):
                t = tgt[pl.ds(pl.multiple_of(rblk * RBLK + g * 16, 16), 16)]
                valid = t != IGNORE_ID
                ts = jnp.where(valid, t, 0)
                logp = (xs_e[g] - ms_e[g]) - _ln(ss_e[g])
                p = jnp.exp(logp)
                av = plsc.load_gather(alo, [ts])
                om = 1.0 - p
                loss = -av * om * om * logp
                acc_e = acc_e + jnp.where(valid, loss, 0.0)
                cnt_e = cnt_e + jnp.where(valid, 1.0, 0.0)
            z = jnp.zeros((16,), jnp.float32)
            neg = jnp.full((16,), _NEG, jnp.float32)
            return (
                tuple(neg for _ in range(NGR)),
                tuple(z for _ in range(NGR)),
                tuple(z for _ in range(NGR)),
                acc_e,
                cnt_e,
            )

        return lax.cond(
            ch == NCH - 1,
            epilogue,
            lambda args: args,
            (tuple(ms_n), tuple(ss_n), tuple(xs_n), acc, cnt),
        )

    z16 = jnp.zeros((16,), jnp.float32)
    neg16 = jnp.full((16,), _NEG, jnp.float32)
    init = (
        tuple(neg16 for _ in range(NGR)),
        tuple(z16 for _ in range(NGR)),
        tuple(z16 for _ in range(NGR)),
        z16,
        z16,
    )
    start_step(0, 0)
    _, _, _, acc, cnt = lax.fori_loop(0, STEPS, step_body, init)
    ob[0, :] = acc
    ob[1, :] = cnt
    pltpu.sync_copy(ob, out_hbm.at[wid])


def _focal_sc(xT, targets, alpha_flat):
    mesh = plsc.VectorSubcoreMesh(core_axis_name="c", subcore_axis_name="s")
    f = pl.kernel(
        _sc_body,
        out_type=jax.ShapeDtypeStruct((NW, 2, 16), jnp.float32),
        mesh=mesh,
        compiler_params=pltpu.CompilerParams(needs_layout_passes=False),
        scratch_types=[
            pltpu.VMEM((2, CCH, RBLK), jnp.float32),
            pltpu.VMEM((RPW,), jnp.int32),
            pltpu.VMEM((C,), jnp.float32),
            pltpu.VMEM((2, 16), jnp.float32),
            pltpu.SemaphoreType.DMA((2,)),
        ],
    )
    return f(xT, targets, alpha_flat)


def _tc_body(x_ref, t_ref, a_ref, loss_ref, cnt_ref):
    xb = x_ref[...]
    t = t_ref[0, 0, :]
    valid = t != IGNORE_ID
    ts = jnp.where(valid, t, 0)
    cls = lax.broadcasted_iota(jnp.int32, (C, BRT), 0)
    onehot = cls == ts[None, :]
    rmax = jnp.max(xb, axis=0)
    ex = jnp.exp(xb - rmax[None, :])
    s = jnp.sum(ex, axis=0)
    xt = jnp.sum(jnp.where(onehot, xb, 0.0), axis=0)
    ab = jnp.broadcast_to(a_ref[:, 0:1], (C, BRT))
    at = jnp.sum(jnp.where(onehot, ab, 0.0), axis=0)
    logp = (xt - rmax) - jnp.log(s)
    p = jnp.exp(logp)
    om = 1.0 - p
    loss = -at * om * om * logp
    vf = valid.astype(jnp.float32)
    loss_ref[0, 0, 0] = jnp.sum(loss * vf)
    cnt_ref[0, 0, 0] = jnp.sum(vf)


def _focal_tc(xT, t3d, alpha_col):
    return pl.pallas_call(
        _tc_body,
        grid=(NB_TC,),
        in_specs=[
            pl.BlockSpec((C, BRT), lambda i: (0, i + N_SC // BRT)),
            pl.BlockSpec((1, 1, BRT), lambda i: (i, 0, 0)),
            pl.BlockSpec((C, 128), lambda i: (0, 0)),
        ],
        out_specs=[
            pl.BlockSpec((1, 1, 1), lambda i: (i, 0, 0), memory_space=pltpu.SMEM),
            pl.BlockSpec((1, 1, 1), lambda i: (i, 0, 0), memory_space=pltpu.SMEM),
        ],
        out_shape=[
            jax.ShapeDtypeStruct((NB_TC, 1, 1), jnp.float32),
            jax.ShapeDtypeStruct((NB_TC, 1, 1), jnp.float32),
        ],
    )(xT, t3d, alpha_col)


@jax.jit
def kernel(inputs, targets, alpha):
    alpha_flat = alpha.reshape(C)
    xT = inputs.T
    t3d = targets[N_SC:].reshape(NB_TC, 1, BRT)
    alpha_col = jnp.broadcast_to(alpha_flat[:, None], (C, 128))
    out = _focal_sc(xT, targets, alpha_flat)
    loss_tc, cnt_tc = _focal_tc(xT, t3d, alpha_col)
    loss_sum = out[:, 0, :].sum() + loss_tc.sum()
    cnt = out[:, 1, :].sum() + cnt_tc.sum()
    return loss_sum / jnp.maximum(cnt, 1.0)


# speedup vs baseline: 1.0328x; 1.0044x over previous
import functools

import jax
import jax.numpy as jnp
from jax import lax
from jax.experimental import pallas as pl
from jax.experimental.pallas import tpu as pltpu
from jax.experimental.pallas import tpu_sc as plsc

N = 16384
C = 1000
GAMMA = 2.0
IGNORE_ID = -1

N_SC = 8192

NC = 2
NS = 16
NW = NC * NS
RPW = N_SC // NW
RBLK = 128
NRB = RPW // RBLK
NGR = RBLK // 16
CCH = 200
NCH = C // CCH
STEPS = NRB * NCH

BRT = 1024
NB_TC = (N - N_SC) // BRT

_LN2 = 0.6931471805599453
_NEG = -3.0e38


def _ln(s):
    bits = lax.bitcast_convert_type(s, jnp.int32)
    e = ((bits >> 23) & 0xFF) - 127
    mbits = (bits & 0x007FFFFF) | 0x3F800000
    m = lax.bitcast_convert_type(mbits, jnp.float32)
    big = m > 1.4142135381698608
    m = jnp.where(big, m * 0.5, m)
    e = e + jnp.where(big, 1, 0)
    z = (m - 1.0) / (m + 1.0)
    z2 = z * z
    poly = 1.0 + z2 * (0.3333333432674408 + z2 * (0.20000000298023224 + z2 * 0.14285714924335480))
    lnm = 2.0 * z * poly
    return e.astype(jnp.float32) * _LN2 + lnm


def _sc_body(xT_hbm, t_hbm, a_hbm, out_hbm, bufs, tgt, alo, ob, sem):
    wid = lax.axis_index("s") * NC + lax.axis_index("c")
    row0 = wid * RPW
    iota = lax.iota(jnp.int32, 16)

    pltpu.sync_copy(t_hbm.at[pl.ds(row0, RPW)], tgt)
    pltpu.sync_copy(a_hbm, alo)

    def step_slices(s):
        rblk = lax.div(s, NCH)
        ch = s - rblk * NCH
        cbase = ch * CCH
        rbase = row0 + rblk * RBLK
        return rblk, ch, cbase, rbase

    def start_step(s, par):
        _, _, cbase, rbase = step_slices(s)
        pltpu.async_copy(
            xT_hbm.at[pl.ds(cbase, CCH), pl.ds(rbase, RBLK)],
            bufs.at[par],
            sem.at[par],
        )

    def wait_step(s, par):
        _, _, cbase, rbase = step_slices(s)
        pltpu.make_async_copy(
            xT_hbm.at[pl.ds(cbase, CCH), pl.ds(rbase, RBLK)],
            bufs.at[par],
            sem.at[par],
        ).wait()

    def step_body(s, carry):
        ms, ss, xs, acc, cnt = carry
        par = lax.rem(s, 2)
        rblk, ch, cbase, _ = step_slices(s)
        wait_step(s, par)

        @pl.when(s + 1 < STEPS)
        def _():
            start_step(s + 1, 1 - par)

        buf = bufs.at[par]

        SLAB = 25
        NSLAB = CCH // SLAB

        def slab_body(j, carry):
            cms_c, css_c = carry
            base = j * SLAB
            cms_o, css_o = [], []
            for g in range(NGR):
                vals = [
                    buf[base + k, pl.ds(g * 16, 16)] for k in range(SLAB)
                ]
                parts = list(vals)
                while len(parts) > 1:
                    parts = [
                        jnp.maximum(parts[q], parts[q + 1])
                        for q in range(0, len(parts) - 1, 2)
                    ] + ([parts[-1]] if len(parts) % 2 else [])
                sm = parts[0]
                sacc = [None, None, None]
                for k, v in enumerate(vals):
                    e = jnp.exp(v - sm)
                    a = sacc[k % 3]
                    sacc[k % 3] = e if a is None else a + e
                ssum = (sacc[0] + sacc[1]) + sacc[2]
                nm = jnp.maximum(cms_c[g], sm)
                ns = css_c[g] * jnp.exp(cms_c[g] - nm) + ssum * jnp.exp(sm - nm)
                cms_o.append(nm)
                css_o.append(ns)
            return tuple(cms_o), tuple(css_o)

        cms, css = lax.fori_loop(
            0, NSLAB, slab_body,
            (
                tuple(jnp.full((16,), _NEG, jnp.float32) for _ in range(NGR)),
                tuple(jnp.zeros((16,), jnp.float32) for _ in range(NGR)),
            ),
        )

        ms_n, ss_n, xs_n = [], [], []
        for g in range(NGR):
            nm = jnp.maximum(ms[g], cms[g])
            ns = ss[g] * jnp.exp(ms[g] - nm) + css[g] * jnp.exp(cms[g] - nm)
            t = tgt[pl.ds(pl.multiple_of(rblk * RBLK + g * 16, 16), 16)]
            ts = jnp.where(t != IGNORE_ID, t, 0)
            inchunk = (ts >= cbase) & (ts < cbase + CCH)
            idx = jnp.clip(ts - cbase, 0, CCH - 1)
            gx = plsc.load_gather(buf, [idx, g * 16 + iota])
            nx = jnp.where(inchunk, gx, xs[g])
            ms_n.append(nm)
            ss_n.append(ns)
            xs_n.append(nx)

        def epilogue(args):
            ms_e, ss_e, xs_e, acc_e, cnt_e = args
            for g in range(NGR):
                t = tgt[pl.ds(pl.multiple_of(rblk * RBLK + g * 16, 16), 16)]
                valid = t != IGNORE_ID
                ts = jnp.where(valid, t, 0)
                logp = (xs_e[g] - ms_e[g]) - _ln(ss_e[g])
                p = jnp.exp(logp)
                av = plsc.load_gather(alo, [ts])
                om = 1.0 - p
                loss = -av * om * om * logp
                acc_e = acc_e + jnp.where(valid, loss, 0.0)
                cnt_e = cnt_e + jnp.where(valid, 1.0, 0.0)
            z = jnp.zeros((16,), jnp.float32)
            neg = jnp.full((16,), _NEG, jnp.float32)
            return (
                tuple(neg for _ in range(NGR)),
                tuple(z for _ in range(NGR)),
                tuple(z for _ in range(NGR)),
                acc_e,
                cnt_e,
            )

        return lax.cond(
            ch == NCH - 1,
            epilogue,
            lambda args: args,
            (tuple(ms_n), tuple(ss_n), tuple(xs_n), acc, cnt),
        )

    z16 = jnp.zeros((16,), jnp.float32)
    neg16 = jnp.full((16,), _NEG, jnp.float32)
    init = (
        tuple(neg16 for _ in range(NGR)),
        tuple(z16 for _ in range(NGR)),
        tuple(z16 for _ in range(NGR)),
        z16,
        z16,
    )
    start_step(0, 0)
    _, _, _, acc, cnt = lax.fori_loop(0, STEPS, step_body, init)
    ob[0, :] = acc
    ob[1, :] = cnt
    pltpu.sync_copy(ob, out_hbm.at[wid])


def _focal_sc(xT, targets, alpha_flat):
    mesh = plsc.VectorSubcoreMesh(core_axis_name="c", subcore_axis_name="s")
    f = pl.kernel(
        _sc_body,
        out_type=jax.ShapeDtypeStruct((NW, 2, 16), jnp.float32),
        mesh=mesh,
        compiler_params=pltpu.CompilerParams(needs_layout_passes=False),
        scratch_types=[
            pltpu.VMEM((2, CCH, RBLK), jnp.float32),
            pltpu.VMEM((RPW,), jnp.int32),
            pltpu.VMEM((C,), jnp.float32),
            pltpu.VMEM((2, 16), jnp.float32),
            pltpu.SemaphoreType.DMA((2,)),
        ],
    )
    return f(xT, targets, alpha_flat)


def _tc_body(x_ref, t_ref, a_ref, loss_ref, cnt_ref):
    xb = x_ref[...]
    t = t_ref[0, 0, :]
    valid = t != IGNORE_ID
    ts = jnp.where(valid, t, 0)
    cls = lax.broadcasted_iota(jnp.int32, (C, BRT), 0)
    onehot = cls == ts[None, :]
    rmax = jnp.max(xb, axis=0)
    ex = jnp.exp(xb - rmax[None, :])
    s = jnp.sum(ex, axis=0)
    xt = jnp.sum(jnp.where(onehot, xb, 0.0), axis=0)
    ab = jnp.broadcast_to(a_ref[:, 0:1], (C, BRT))
    at = jnp.sum(jnp.where(onehot, ab, 0.0), axis=0)
    logp = (xt - rmax) - jnp.log(s)
    p = jnp.exp(logp)
    om = 1.0 - p
    loss = -at * om * om * logp
    vf = valid.astype(jnp.float32)
    loss_ref[0, 0, 0] = jnp.sum(loss * vf)
    cnt_ref[0, 0, 0] = jnp.sum(vf)


def _focal_tc(xT, t3d, alpha_col):
    return pl.pallas_call(
        _tc_body,
        grid=(NB_TC,),
        in_specs=[
            pl.BlockSpec((C, BRT), lambda i: (0, i + N_SC // BRT)),
            pl.BlockSpec((1, 1, BRT), lambda i: (i, 0, 0)),
            pl.BlockSpec((C, 128), lambda i: (0, 0)),
        ],
        out_specs=[
            pl.BlockSpec((1, 1, 1), lambda i: (i, 0, 0), memory_space=pltpu.SMEM),
            pl.BlockSpec((1, 1, 1), lambda i: (i, 0, 0), memory_space=pltpu.SMEM),
        ],
        out_shape=[
            jax.ShapeDtypeStruct((NB_TC, 1, 1), jnp.float32),
            jax.ShapeDtypeStruct((NB_TC, 1, 1), jnp.float32),
        ],
    )(xT, t3d, alpha_col)


@jax.jit
def kernel(inputs, targets, alpha):
    alpha_flat = alpha.reshape(C)
    xT = inputs.T
    t3d = targets[N_SC:].reshape(NB_TC, 1, BRT)
    alpha_col = jnp.broadcast_to(alpha_flat[:, None], (C, 128))
    out = _focal_sc(xT, targets, alpha_flat)
    loss_tc, cnt_tc = _focal_tc(xT, t3d, alpha_col)
    loss_sum = out[:, 0, :].sum() + loss_tc.sum()
    cnt = out[:, 1, :].sum() + cnt_tc.sum()
    return loss_sum / jnp.maximum(cnt, 1.0)


# speedup vs baseline: 1.0964x; 1.0616x over previous
import functools

import jax
import jax.numpy as jnp
from jax import lax
from jax.experimental import pallas as pl
from jax.experimental.pallas import tpu as pltpu
from jax.experimental.pallas import tpu_sc as plsc

N = 16384
C = 1000
GAMMA = 2.0
IGNORE_ID = -1

N_SC = 8192

NC = 2
NS = 16
NW = NC * NS
RPW = N_SC // NW
RBLK = 128
NRB = RPW // RBLK
NGR = RBLK // 16
CCH = 200
NCH = C // CCH
STEPS = NRB * NCH

BRT = 1024
NB_TC = (N - N_SC) // BRT

_LN2 = 0.6931471805599453
_NEG = -3.0e38


def _ln(s):
    bits = lax.bitcast_convert_type(s, jnp.int32)
    e = ((bits >> 23) & 0xFF) - 127
    mbits = (bits & 0x007FFFFF) | 0x3F800000
    m = lax.bitcast_convert_type(mbits, jnp.float32)
    big = m > 1.4142135381698608
    m = jnp.where(big, m * 0.5, m)
    e = e + jnp.where(big, 1, 0)
    z = (m - 1.0) / (m + 1.0)
    z2 = z * z
    poly = 1.0 + z2 * (0.3333333432674408 + z2 * (0.20000000298023224 + z2 * 0.14285714924335480))
    lnm = 2.0 * z * poly
    return e.astype(jnp.float32) * _LN2 + lnm


def _sc_body(xT_hbm, t_hbm, a_hbm, out_hbm, bufs, tgt, alo, ob, sem):
    wid = lax.axis_index("s") * NC + lax.axis_index("c")
    row0 = wid * RPW
    iota = lax.iota(jnp.int32, 16)

    pltpu.sync_copy(t_hbm.at[pl.ds(row0, RPW)], tgt)
    pltpu.sync_copy(a_hbm, alo)

    def step_slices(s):
        rblk = lax.div(s, NCH)
        ch = s - rblk * NCH
        cbase = ch * CCH
        rbase = row0 + rblk * RBLK
        return rblk, ch, cbase, rbase

    def start_step(s, par):
        _, _, cbase, rbase = step_slices(s)
        pltpu.async_copy(
            xT_hbm.at[pl.ds(cbase, CCH), pl.ds(rbase, RBLK)],
            bufs.at[par],
            sem.at[par],
        )

    def wait_step(s, par):
        _, _, cbase, rbase = step_slices(s)
        pltpu.make_async_copy(
            xT_hbm.at[pl.ds(cbase, CCH), pl.ds(rbase, RBLK)],
            bufs.at[par],
            sem.at[par],
        ).wait()

    def step_body(s, carry):
        ms, ss, xs, acc, cnt = carry
        par = lax.rem(s, 2)
        rblk, ch, cbase, _ = step_slices(s)
        wait_step(s, par)

        @pl.when(s + 1 < STEPS)
        def _():
            start_step(s + 1, 1 - par)

        buf = bufs.at[par]

        SLAB = 25
        NSLAB = CCH // SLAB

        def slab_body(j, carry):
            cms_c, css_c = carry
            base = j * SLAB
            cms_o, css_o = [], []
            for g in range(NGR):
                vals = [
                    buf[base + k, pl.ds(g * 16, 16)] for k in range(SLAB)
                ]
                parts = list(vals)
                while len(parts) > 1:
                    parts = [
                        jnp.maximum(parts[q], parts[q + 1])
                        for q in range(0, len(parts) - 1, 2)
                    ] + ([parts[-1]] if len(parts) % 2 else [])
                sm = parts[0]
                sacc = [None, None, None]
                for k, v in enumerate(vals):
                    e = jnp.exp(v - sm)
                    a = sacc[k % 3]
                    sacc[k % 3] = e if a is None else a + e
                ssum = (sacc[0] + sacc[1]) + sacc[2]
                nm = jnp.maximum(cms_c[g], sm)
                ns = css_c[g] * jnp.exp(cms_c[g] - nm) + ssum * jnp.exp(sm - nm)
                cms_o.append(nm)
                css_o.append(ns)
            return tuple(cms_o), tuple(css_o)

        cms, css = lax.fori_loop(
            0, NSLAB, slab_body,
            (
                tuple(jnp.full((16,), _NEG, jnp.float32) for _ in range(NGR)),
                tuple(jnp.zeros((16,), jnp.float32) for _ in range(NGR)),
            ),
        )

        ms_n, ss_n, xs_n = [], [], []
        for g in range(NGR):
            nm = jnp.maximum(ms[g], cms[g])
            ns = ss[g] * jnp.exp(ms[g] - nm) + css[g] * jnp.exp(cms[g] - nm)
            t = tgt[pl.ds(pl.multiple_of(rblk * RBLK + g * 16, 16), 16)]
            ts = jnp.where(t != IGNORE_ID, t, 0)
            inchunk = (ts >= cbase) & (ts < cbase + CCH)
            idx = jnp.clip(ts - cbase, 0, CCH - 1)
            gx = plsc.load_gather(buf, [idx, g * 16 + iota])
            nx = jnp.where(inchunk, gx, xs[g])
            ms_n.append(nm)
            ss_n.append(ns)
            xs_n.append(nx)

        def epilogue(args):
            ms_e, ss_e, xs_e, acc_e, cnt_e = args
            for g in range(NGR):
                t = tgt[pl.ds(pl.multiple_of(rblk * RBLK + g * 16, 16), 16)]
                valid = t != IGNORE_ID
                ts = jnp.where(valid, t, 0)
                logp = (xs_e[g] - ms_e[g]) - _ln(ss_e[g])
                p = jnp.exp(logp)
                av = plsc.load_gather(alo, [ts])
                om = 1.0 - p
                loss = -av * om * om * logp
                acc_e = acc_e + jnp.where(valid, loss, 0.0)
                cnt_e = cnt_e + jnp.where(valid, 1.0, 0.0)
            z = jnp.zeros((16,), jnp.float32)
            neg = jnp.full((16,), _NEG, jnp.float32)
            return (
                tuple(neg for _ in range(NGR)),
                tuple(z for _ in range(NGR)),
                tuple(z for _ in range(NGR)),
                acc_e,
                cnt_e,
            )

        return lax.cond(
            ch == NCH - 1,
            epilogue,
            lambda args: args,
            (tuple(ms_n), tuple(ss_n), tuple(xs_n), acc, cnt),
        )

    z16 = jnp.zeros((16,), jnp.float32)
    neg16 = jnp.full((16,), _NEG, jnp.float32)
    init = (
        tuple(neg16 for _ in range(NGR)),
        tuple(z16 for _ in range(NGR)),
        tuple(z16 for _ in range(NGR)),
        z16,
        z16,
    )
    start_step(0, 0)
    _, _, _, acc, cnt = lax.fori_loop(0, STEPS, step_body, init)
    ob[0, :] = acc
    ob[1, :] = cnt
    pltpu.sync_copy(ob, out_hbm.at[wid])


def _focal_sc(xT, targets, alpha_flat):
    mesh = plsc.VectorSubcoreMesh(core_axis_name="c", subcore_axis_name="s")
    f = pl.kernel(
        _sc_body,
        out_type=jax.ShapeDtypeStruct((NW, 2, 16), jnp.float32),
        mesh=mesh,
        compiler_params=pltpu.CompilerParams(needs_layout_passes=False),
        scratch_types=[
            pltpu.VMEM((2, CCH, RBLK), jnp.float32),
            pltpu.VMEM((RPW,), jnp.int32),
            pltpu.VMEM((C,), jnp.float32),
            pltpu.VMEM((2, 16), jnp.float32),
            pltpu.SemaphoreType.DMA((2,)),
        ],
    )
    return f(xT, targets, alpha_flat)


def _tc_body(x_ref, t_ref, a_ref, loss_ref, cnt_ref):
    xb = x_ref[...]
    t = t_ref[0, 0, :]
    valid = t != IGNORE_ID
    ts = jnp.where(valid, t, 0)
    cls = lax.broadcasted_iota(jnp.int32, (C, BRT), 0)
    onehot = cls == ts[None, :]
    rmax = jnp.max(xb, axis=0)
    ex = jnp.exp(xb - rmax[None, :])
    s = jnp.sum(ex, axis=0)
    xt = jnp.sum(jnp.where(onehot, xb, 0.0), axis=0)
    ab = jnp.broadcast_to(a_ref[:, 0:1], (C, BRT))
    at = jnp.sum(jnp.where(onehot, ab, 0.0), axis=0)
    logp = (xt - rmax) - jnp.log(s)
    p = jnp.exp(logp)
    om = 1.0 - p
    loss = -at * om * om * logp
    vf = valid.astype(jnp.float32)
    loss_ref[0, 0, 0] = jnp.sum(loss * vf)
    cnt_ref[0, 0, 0] = jnp.sum(vf)


def _focal_tc(xT, t3d, alpha_col):
    return pl.pallas_call(
        _tc_body,
        grid=(NB_TC,),
        in_specs=[
            pl.BlockSpec((C, BRT), lambda i: (0, i + N_SC // BRT)),
            pl.BlockSpec((1, 1, BRT), lambda i: (i, 0, 0)),
            pl.BlockSpec((C, 128), lambda i: (0, 0)),
        ],
        out_specs=[
            pl.BlockSpec((1, 1, 1), lambda i: (i, 0, 0), memory_space=pltpu.SMEM),
            pl.BlockSpec((1, 1, 1), lambda i: (i, 0, 0), memory_space=pltpu.SMEM),
        ],
        out_shape=[
            jax.ShapeDtypeStruct((NB_TC, 1, 1), jnp.float32),
            jax.ShapeDtypeStruct((NB_TC, 1, 1), jnp.float32),
        ],
    )(xT, t3d, alpha_col)


def _combine_body(osc_ref, ltc_ref, ctc_ref, out_ref):
    loss = jnp.sum(osc_ref[:, 0, :]) + jnp.sum(ltc_ref[...])
    cnt = jnp.sum(osc_ref[:, 1, :]) + jnp.sum(ctc_ref[...])
    out_ref[0, 0] = loss / jnp.maximum(cnt, 1.0)


def _combine(out_sc, loss_tc, cnt_tc):
    return pl.pallas_call(
        _combine_body,
        out_specs=pl.BlockSpec(memory_space=pltpu.SMEM),
        out_shape=jax.ShapeDtypeStruct((1, 1), jnp.float32),
    )(out_sc, loss_tc, cnt_tc)


@jax.jit
def kernel(inputs, targets, alpha):
    alpha_flat = alpha.reshape(C)
    xT = inputs.T
    t3d = targets[N_SC:].reshape(NB_TC, 1, BRT)
    alpha_col = jnp.broadcast_to(alpha_flat[:, None], (C, 128))
    out = _focal_sc(xT, targets, alpha_flat)
    loss_tc, cnt_tc = _focal_tc(xT, t3d, alpha_col)
    return _combine(out, loss_tc, cnt_tc)[0, 0]
